# Initial kernel scaffold; baseline (speedup 1.0000x reference)
import jax
import jax.numpy as jnp
from jax.experimental import pallas as pl


def kernel(x, edge_gate_type, edge_index, gate_type_embed, Wr, br, W1, b1, W2, b2, gamma, beta):
    raise NotImplementedError("write your pallas kernel here")


# speedup vs baseline: 18.3357x; 18.3357x over previous
import functools

import jax
import jax.numpy as jnp
from jax import lax
from jax.experimental import pallas as pl
from jax.experimental.pallas import tpu as pltpu
from jax.experimental.pallas import tpu_sc as plsc

N = 10000
E = 320000
D = 128
NE = 8
NGT = 20
H = 2 * D

NC = 2
NS = 16
NW = NC * NS
EW = E // NW
VECS = EW // 16
ROWS = (EW + 127) // 128
HB = N * NGT
ZB = 12608
HP = NS * ZB
TRASH = HB
COPY_CHUNK = 8000


def _sc_hist_body(dst_hbm, egt_hbm, zeros_hbm, out_hbm,
                  dst_v, egt_v, idx_v, ones_v, zbuf_v, hist_sh, sem):
    cid = lax.axis_index("c")
    sid = lax.axis_index("s")
    wid = cid * NS + sid

    pltpu.sync_copy(zeros_hbm, zbuf_v)
    pltpu.sync_copy(zbuf_v, hist_sh.at[pl.ds(sid * ZB, ZB)])

    base = wid * EW
    pltpu.sync_copy(dst_hbm.at[pl.ds(base, EW)], dst_v)
    pltpu.sync_copy(egt_hbm.at[pl.ds(base, EW)], egt_v)

    for j in range(8):
        ones_v[pl.ds(j * 16, 16)] = jnp.ones((16,), jnp.float32)

    def body(i, carry):
        d = dst_v[pl.ds(i * 16, 16)]
        g = egt_v[pl.ds(i * 16, 16)]
        r = i // 8
        c = (i % 8) * 16
        idx_v[r, pl.ds(c, 16)] = d * NGT + g
        return carry

    lax.fori_loop(0, VECS, body, 0)
    for j in range(7):
        idx_v[ROWS - 1, pl.ds(16 + j * 16, 16)] = jnp.full(
            (16,), TRASH, jnp.int32)

    plsc.subcore_barrier()
    for r in range(ROWS):
        pltpu.sync_copy(ones_v, hist_sh.at[idx_v.at[r]], add=True)
    plsc.subcore_barrier()

    obase = cid * HB
    cbuf = zbuf_v.at[pl.ds(0, COPY_CHUNK)]
    off0 = sid * COPY_CHUNK
    pltpu.sync_copy(hist_sh.at[pl.ds(off0, COPY_CHUNK)], cbuf)
    pltpu.sync_copy(cbuf, out_hbm.at[pl.ds(obase + off0, COPY_CHUNK)])

    @pl.when(sid < 9)
    def _():
        off = (sid + 16) * COPY_CHUNK
        pltpu.sync_copy(hist_sh.at[pl.ds(off, COPY_CHUNK)], cbuf)
        pltpu.sync_copy(cbuf, out_hbm.at[pl.ds(obase + off, COPY_CHUNK)])


@functools.cache
def _sc_hist():
    return pl.kernel(
        _sc_hist_body,
        out_type=jax.ShapeDtypeStruct((NC * HB,), jnp.float32),
        mesh=plsc.VectorSubcoreMesh(core_axis_name="c", subcore_axis_name="s",
                                    num_cores=NC, num_subcores=NS),
        scratch_types=[
            pltpu.VMEM((EW,), jnp.int32),
            pltpu.VMEM((EW,), jnp.int32),
            pltpu.VMEM((ROWS, 128), jnp.int32),
            pltpu.VMEM((128,), jnp.float32),
            pltpu.VMEM((ZB,), jnp.float32),
            pltpu.VMEM_SHARED((HP,), jnp.float32),
            pltpu.SemaphoreType.DMA,
        ],
    )


def _tc_body(x_ref, h2_ref, gte_ref, wr_ref, br_ref, w1_ref, b1_ref,
             w2_ref, b2_ref, gamma_ref, beta_ref, o_ref):
    x = x_ref[...]
    counts = h2_ref[0] + h2_ref[1]
    deg = jnp.sum(counts, axis=1, keepdims=True)
    gate_logits = counts[:, 0:1] * gte_ref[0:1, :]
    for t in range(1, NGT):
        gate_logits = gate_logits + counts[:, t:t + 1] * gte_ref[t:t + 1, :]
    gate_logits = gate_logits / jnp.maximum(deg, 1.0)
    logits = (jnp.dot(x, wr_ref[...], preferred_element_type=jnp.float32)
              + br_ref[...] + gate_logits)

    iota = lax.broadcasted_iota(jnp.int32, logits.shape, 1)
    m1 = jnp.max(logits, axis=1, keepdims=True)
    idx1 = jnp.min(jnp.where(logits == m1, iota, NE), axis=1, keepdims=True)
    masked = jnp.where(iota == idx1, -jnp.inf, logits)
    m2 = jnp.max(masked, axis=1, keepdims=True)
    idx2 = jnp.min(jnp.where(masked == m2, iota, NE), axis=1, keepdims=True)
    g1 = 1.0 / (1.0 + jnp.exp(m2 - m1))
    w = (jnp.where(iota == idx1, g1, 0.0)
         + jnp.where(iota == idx2, 1.0 - g1, 0.0))

    inv_sqrt2 = 0.7071067811865476
    acc = jnp.zeros_like(x)
    for e in range(NE):
        h = jnp.dot(x, w1_ref[e], preferred_element_type=jnp.float32)
        h = h + b1_ref[e:e + 1]
        h = h * 0.5 * (1.0 + lax.erf(h * inv_sqrt2))
        oe = jnp.dot(h, w2_ref[e], preferred_element_type=jnp.float32)
        oe = oe + b2_ref[e:e + 1]
        acc = acc + w[:, e:e + 1] * oe

    mu = jnp.mean(acc, axis=1, keepdims=True)
    var = jnp.mean((acc - mu) ** 2, axis=1, keepdims=True)
    o_ref[...] = ((acc - mu) * lax.rsqrt(var + 1e-5) * gamma_ref[...]
                  + beta_ref[...])


def kernel(x, edge_gate_type, edge_index, gate_type_embed, Wr, br,
           W1, b1, W2, b2, gamma, beta):
    dst = edge_index[1].astype(jnp.int32)
    egt = edge_gate_type.astype(jnp.int32)
    zeros = jnp.zeros((ZB,), jnp.float32)

    hist = _sc_hist()(dst, egt, zeros)
    h2 = hist.reshape(NC, N, NGT)

    Bn = 1000
    grid = (N // Bn,)
    out = pl.pallas_call(
        _tc_body,
        grid=grid,
        in_specs=[
            pl.BlockSpec((Bn, D), lambda i: (i, 0)),
            pl.BlockSpec((NC, Bn, NGT), lambda i: (0, i, 0)),
            pl.BlockSpec((NGT, NE), lambda i: (0, 0)),
            pl.BlockSpec((D, NE), lambda i: (0, 0)),
            pl.BlockSpec((1, NE), lambda i: (0, 0)),
            pl.BlockSpec((NE, D, H), lambda i: (0, 0, 0)),
            pl.BlockSpec((NE, H), lambda i: (0, 0)),
            pl.BlockSpec((NE, H, D), lambda i: (0, 0, 0)),
            pl.BlockSpec((NE, D), lambda i: (0, 0)),
            pl.BlockSpec((1, D), lambda i: (0, 0)),
            pl.BlockSpec((1, D), lambda i: (0, 0)),
        ],
        out_specs=pl.BlockSpec((Bn, D), lambda i: (i, 0)),
        out_shape=jax.ShapeDtypeStruct((N, D), jnp.float32),
        compiler_params=pltpu.CompilerParams(
            dimension_semantics=("arbitrary",)),
    )(x, h2, gate_type_embed, Wr, br.reshape(1, NE), W1, b1, W2, b2,
      gamma.reshape(1, D), beta.reshape(1, D))
    return out


# speedup vs baseline: 19.2858x; 1.0518x over previous
import functools

import jax
import jax.numpy as jnp
from jax import lax
from jax.experimental import pallas as pl
from jax.experimental.pallas import tpu as pltpu
from jax.experimental.pallas import tpu_sc as plsc

N = 10000
E = 320000
D = 128
NE = 8
NGT = 20
H = 2 * D

NC = 2
NS = 16
NW = NC * NS
EW = E // NW
VECS = EW // 16
ROWS = (EW + 127) // 128
HB = N * NGT
ZB = 12608
HP = NS * ZB
TRASH = HB
COPY_CHUNK = 8000


def _sc_hist_body(dst_hbm, egt_hbm, zeros_hbm, out_hbm,
                  dst_v, egt_v, idx_v, ones_v, zbuf_v, hist_sh, sem):
    cid = lax.axis_index("c")
    sid = lax.axis_index("s")
    wid = cid * NS + sid

    pltpu.sync_copy(zeros_hbm, zbuf_v)
    pltpu.sync_copy(zbuf_v, hist_sh.at[pl.ds(sid * ZB, ZB)])

    base = wid * EW
    cp_d = pltpu.async_copy(dst_hbm.at[pl.ds(base, EW)], dst_v, sem)
    cp_g = pltpu.async_copy(egt_hbm.at[pl.ds(base, EW)], egt_v, sem)

    for j in range(8):
        ones_v[pl.ds(j * 16, 16)] = jnp.ones((16,), jnp.float32)
    cp_d.wait()
    cp_g.wait()

    def body(r, carry):
        for j in range(8):
            off = r * 128 + j * 16
            d = dst_v[pl.ds(off, 16)]
            g = egt_v[pl.ds(off, 16)]
            idx_v[r, pl.ds(j * 16, 16)] = d * NGT + g
        return carry

    lax.fori_loop(0, ROWS - 1, body, 0)
    off = (ROWS - 1) * 128
    idx_v[ROWS - 1, pl.ds(0, 16)] = (dst_v[pl.ds(off, 16)] * NGT
                                     + egt_v[pl.ds(off, 16)])
    for j in range(7):
        idx_v[ROWS - 1, pl.ds(16 + j * 16, 16)] = jnp.full(
            (16,), TRASH, jnp.int32)

    plsc.subcore_barrier()
    copies = [pltpu.async_copy(ones_v, hist_sh.at[idx_v.at[r]], sem, add=True)
              for r in range(ROWS)]
    for c in copies:
        c.wait()
    plsc.subcore_barrier()

    obase = cid * HB
    cbuf = zbuf_v.at[pl.ds(0, COPY_CHUNK)]
    off0 = sid * COPY_CHUNK
    pltpu.sync_copy(hist_sh.at[pl.ds(off0, COPY_CHUNK)], cbuf)
    pltpu.sync_copy(cbuf, out_hbm.at[pl.ds(obase + off0, COPY_CHUNK)])

    @pl.when(sid < 9)
    def _():
        off = (sid + 16) * COPY_CHUNK
        pltpu.sync_copy(hist_sh.at[pl.ds(off, COPY_CHUNK)], cbuf)
        pltpu.sync_copy(cbuf, out_hbm.at[pl.ds(obase + off, COPY_CHUNK)])


@functools.cache
def _sc_hist():
    return pl.kernel(
        _sc_hist_body,
        out_type=jax.ShapeDtypeStruct((NC * HB,), jnp.float32),
        mesh=plsc.VectorSubcoreMesh(core_axis_name="c", subcore_axis_name="s",
                                    num_cores=NC, num_subcores=NS),
        scratch_types=[
            pltpu.VMEM((EW,), jnp.int32),
            pltpu.VMEM((EW,), jnp.int32),
            pltpu.VMEM((ROWS, 128), jnp.int32),
            pltpu.VMEM((128,), jnp.float32),
            pltpu.VMEM((ZB,), jnp.float32),
            pltpu.VMEM_SHARED((HP,), jnp.float32),
            pltpu.SemaphoreType.DMA,
        ],
    )


def _tc_body(x_ref, h2_ref, gte_ref, wr_ref, br_ref, w1c_ref, b1c_ref,
             w2c_ref, b2_ref, sel_ref, gamma_ref, beta_ref, o_ref):
    x = x_ref[...]
    counts = h2_ref[0] + h2_ref[1]
    deg = jnp.sum(counts, axis=1, keepdims=True)
    gate_logits = lax.dot_general(
        counts, gte_ref[...], (((1,), (0,)), ((), ())),
        precision=lax.Precision.HIGHEST,
        preferred_element_type=jnp.float32)
    gate_logits = gate_logits / jnp.maximum(deg, 1.0)
    logits = (jnp.dot(x, wr_ref[...], preferred_element_type=jnp.float32)
              + br_ref[...] + gate_logits)

    iota = lax.broadcasted_iota(jnp.int32, logits.shape, 1)
    m1 = jnp.max(logits, axis=1, keepdims=True)
    idx1 = jnp.min(jnp.where(logits == m1, iota, NE), axis=1, keepdims=True)
    masked = jnp.where(iota == idx1, -jnp.inf, logits)
    m2 = jnp.max(masked, axis=1, keepdims=True)
    idx2 = jnp.min(jnp.where(masked == m2, iota, NE), axis=1, keepdims=True)
    g1 = 1.0 / (1.0 + jnp.exp(m2 - m1))
    w = (jnp.where(iota == idx1, g1, 0.0)
         + jnp.where(iota == idx2, 1.0 - g1, 0.0))

    inv_sqrt2 = 0.7071067811865476
    hall = (jnp.dot(x, w1c_ref[...], preferred_element_type=jnp.float32)
            + b1c_ref[...])
    wrep = jnp.dot(w, sel_ref[...],
                   preferred_element_type=jnp.float32)
    g = (hall * wrep) * (1.0 + lax.erf(hall * inv_sqrt2))
    acc = (jnp.dot(g, w2c_ref[...], preferred_element_type=jnp.float32)
           + jnp.dot(w, b2_ref[...], preferred_element_type=jnp.float32))

    mu = jnp.mean(acc, axis=1, keepdims=True)
    var = jnp.mean((acc - mu) ** 2, axis=1, keepdims=True)
    o_ref[...] = ((acc - mu) * lax.rsqrt(var + 1e-5) * gamma_ref[...]
                  + beta_ref[...])


def kernel(x, edge_gate_type, edge_index, gate_type_embed, Wr, br,
           W1, b1, W2, b2, gamma, beta):
    dst = edge_index[1].astype(jnp.int32)
    egt = edge_gate_type.astype(jnp.int32)
    zeros = jnp.zeros((ZB,), jnp.float32)

    hist = _sc_hist()(dst, egt, zeros)
    h2 = hist.reshape(NC, N, NGT)

    HH = NE * H
    W1c = W1.transpose(1, 0, 2).reshape(D, HH)
    b1c = b1.reshape(1, HH)
    W2c = W2.reshape(HH, D)
    sel = 0.5 * jnp.repeat(jnp.eye(NE, dtype=jnp.float32), H, axis=1)

    Bn = 1000
    grid = (N // Bn,)
    out = pl.pallas_call(
        _tc_body,
        grid=grid,
        in_specs=[
            pl.BlockSpec((Bn, D), lambda i: (i, 0)),
            pl.BlockSpec((NC, Bn, NGT), lambda i: (0, i, 0)),
            pl.BlockSpec((NGT, NE), lambda i: (0, 0)),
            pl.BlockSpec((D, NE), lambda i: (0, 0)),
            pl.BlockSpec((1, NE), lambda i: (0, 0)),
            pl.BlockSpec((D, HH), lambda i: (0, 0)),
            pl.BlockSpec((1, HH), lambda i: (0, 0)),
            pl.BlockSpec((HH, D), lambda i: (0, 0)),
            pl.BlockSpec((NE, D), lambda i: (0, 0)),
            pl.BlockSpec((NE, HH), lambda i: (0, 0)),
            pl.BlockSpec((1, D), lambda i: (0, 0)),
            pl.BlockSpec((1, D), lambda i: (0, 0)),
        ],
        out_specs=pl.BlockSpec((Bn, D), lambda i: (i, 0)),
        out_shape=jax.ShapeDtypeStruct((N, D), jnp.float32),
        compiler_params=pltpu.CompilerParams(
            dimension_semantics=("arbitrary",)),
    )(x, h2, gate_type_embed, Wr, br.reshape(1, NE), W1c, b1c, W2c, b2,
      sel, gamma.reshape(1, D), beta.reshape(1, D))
    return out


# speedup vs baseline: 21.2297x; 1.1008x over previous
import functools

import jax
import jax.numpy as jnp
from jax import lax
from jax.experimental import pallas as pl
from jax.experimental.pallas import tpu as pltpu
from jax.experimental.pallas import tpu_sc as plsc

N = 10000
E = 320000
D = 128
NE = 8
NGT = 20
H = 2 * D

NP = 10240
NC = 2
NS = 16
NW = NC * NS
EW = E // NW
ROWS = (EW + 127) // 128
HB = NGT * NP
ZB = HB // NS
TRASH = N


def _sc_hist_body(dst_hbm, egt_hbm, zeros_hbm, out_hbm,
                  dst_v, egt_v, idx_v, ones_v, zbuf_v, hist_sh, sem):
    cid = lax.axis_index("c")
    sid = lax.axis_index("s")
    wid = cid * NS + sid

    pltpu.sync_copy(zeros_hbm, zbuf_v)
    pltpu.sync_copy(zbuf_v, hist_sh.at[pl.ds(sid * ZB, ZB)])

    base = wid * EW
    cp_d = pltpu.async_copy(dst_hbm.at[pl.ds(base, EW)], dst_v, sem)
    cp_g = pltpu.async_copy(egt_hbm.at[pl.ds(base, EW)], egt_v, sem)

    for j in range(8):
        ones_v[pl.ds(j * 16, 16)] = jnp.ones((16,), jnp.float32)
    cp_d.wait()
    cp_g.wait()

    def body(r, carry):
        for j in range(8):
            off = r * 128 + j * 16
            d = dst_v[pl.ds(off, 16)]
            g = egt_v[pl.ds(off, 16)]
            idx_v[r, pl.ds(j * 16, 16)] = g * NP + d
        return carry

    lax.fori_loop(0, ROWS - 1, body, 0)
    off = (ROWS - 1) * 128
    idx_v[ROWS - 1, pl.ds(0, 16)] = (egt_v[pl.ds(off, 16)] * NP
                                     + dst_v[pl.ds(off, 16)])
    for j in range(7):
        idx_v[ROWS - 1, pl.ds(16 + j * 16, 16)] = jnp.full(
            (16,), TRASH, jnp.int32)

    plsc.subcore_barrier()
    copies = [pltpu.async_copy(ones_v, hist_sh.at[idx_v.at[r]], sem, add=True)
              for r in range(ROWS)]
    for c in copies:
        c.wait()
    plsc.subcore_barrier()

    off0 = sid * ZB
    pltpu.sync_copy(hist_sh.at[pl.ds(off0, ZB)], zbuf_v)
    pltpu.sync_copy(zbuf_v, out_hbm.at[pl.ds(cid * HB + off0, ZB)])


@functools.cache
def _sc_hist():
    return pl.kernel(
        _sc_hist_body,
        out_type=jax.ShapeDtypeStruct((NC * HB,), jnp.float32),
        mesh=plsc.VectorSubcoreMesh(core_axis_name="c", subcore_axis_name="s",
                                    num_cores=NC, num_subcores=NS),
        scratch_types=[
            pltpu.VMEM((EW,), jnp.int32),
            pltpu.VMEM((EW,), jnp.int32),
            pltpu.VMEM((ROWS, 128), jnp.int32),
            pltpu.VMEM((128,), jnp.float32),
            pltpu.VMEM((ZB,), jnp.float32),
            pltpu.VMEM_SHARED((HB,), jnp.float32),
            pltpu.SemaphoreType.DMA,
        ],
    )


def _tc_body(x_ref, ht_ref, gte_ref, wr_ref, br_ref, w1_ref, b1c_ref,
             w2c_ref, b2_ref, sel_ref, gamma_ref, beta_ref, o_ref):
    x = x_ref[...]
    ct = ht_ref[0:NGT] + ht_ref[NGT:2 * NGT]
    gl_aug = lax.dot_general(
        ct, gte_ref[...], (((0,), (0,)), ((), ())),
        precision=lax.Precision.HIGHEST,
        preferred_element_type=jnp.float32)
    deg = gl_aug[:, NE:NE + 1]
    gate_logits = gl_aug[:, 0:NE] / jnp.maximum(deg, 1.0)
    logits = (jnp.dot(x, wr_ref[...], preferred_element_type=jnp.float32)
              + br_ref[...] + gate_logits)

    iota = lax.broadcasted_iota(jnp.int32, logits.shape, 1)
    m1 = jnp.max(logits, axis=1, keepdims=True)
    idx1 = jnp.min(jnp.where(logits == m1, iota, NE), axis=1, keepdims=True)
    masked = jnp.where(iota == idx1, -jnp.inf, logits)
    m2 = jnp.max(masked, axis=1, keepdims=True)
    idx2 = jnp.min(jnp.where(masked == m2, iota, NE), axis=1, keepdims=True)
    g1 = 1.0 / (1.0 + jnp.exp(m2 - m1))
    w = (jnp.where(iota == idx1, g1, 0.0)
         + jnp.where(iota == idx2, 1.0 - g1, 0.0))

    inv_sqrt2 = 0.7071067811865476
    xb = x.astype(jnp.bfloat16)
    hall = jnp.concatenate(
        [jnp.dot(xb, w1_ref[e], preferred_element_type=jnp.float32)
         for e in range(NE)], axis=1) + b1c_ref[...]
    wrep = jnp.dot(w, sel_ref[...],
                   preferred_element_type=jnp.float32)
    g = (hall * wrep) * (1.0 + lax.erf(hall * inv_sqrt2))
    acc = (jnp.dot(g.astype(jnp.bfloat16), w2c_ref[...],
                   preferred_element_type=jnp.float32)
           + jnp.dot(w, b2_ref[...], preferred_element_type=jnp.float32))

    mu = jnp.mean(acc, axis=1, keepdims=True)
    var = jnp.mean((acc - mu) ** 2, axis=1, keepdims=True)
    o_ref[...] = ((acc - mu) * lax.rsqrt(var + 1e-5) * gamma_ref[...]
                  + beta_ref[...])


def kernel(x, edge_gate_type, edge_index, gate_type_embed, Wr, br,
           W1, b1, W2, b2, gamma, beta):
    dst = edge_index[1].astype(jnp.int32)
    egt = edge_gate_type.astype(jnp.int32)
    zeros = jnp.zeros((ZB,), jnp.float32)

    hist = _sc_hist()(dst, egt, zeros)
    ht = hist.reshape(NC * NGT, NP)

    HH = NE * H
    W1b = W1.astype(jnp.bfloat16)
    b1c = b1.reshape(1, HH)
    W2cb = W2.reshape(HH, D).astype(jnp.bfloat16)
    sel = 0.5 * jnp.repeat(jnp.eye(NE, dtype=jnp.float32), H, axis=1)
    gte_aug = jnp.concatenate(
        [gate_type_embed, jnp.ones((NGT, 1), jnp.float32)], axis=1)

    Bn = 512
    grid = ((N + Bn - 1) // Bn,)
    out = pl.pallas_call(
        _tc_body,
        grid=grid,
        in_specs=[
            pl.BlockSpec((Bn, D), lambda i: (i, 0)),
            pl.BlockSpec((NC * NGT, Bn), lambda i: (0, i)),
            pl.BlockSpec((NGT, NE + 1), lambda i: (0, 0)),
            pl.BlockSpec((D, NE), lambda i: (0, 0)),
            pl.BlockSpec((1, NE), lambda i: (0, 0)),
            pl.BlockSpec((NE, D, H), lambda i: (0, 0, 0)),
            pl.BlockSpec((1, HH), lambda i: (0, 0)),
            pl.BlockSpec((HH, D), lambda i: (0, 0)),
            pl.BlockSpec((NE, D), lambda i: (0, 0)),
            pl.BlockSpec((NE, HH), lambda i: (0, 0)),
            pl.BlockSpec((1, D), lambda i: (0, 0)),
            pl.BlockSpec((1, D), lambda i: (0, 0)),
        ],
        out_specs=pl.BlockSpec((Bn, D), lambda i: (i, 0)),
        out_shape=jax.ShapeDtypeStruct((N, D), jnp.float32),
        compiler_params=pltpu.CompilerParams(
            dimension_semantics=("arbitrary",)),
    )(x, ht, gte_aug, Wr, br.reshape(1, NE), W1b, b1c,
      W2cb, b2, sel, gamma.reshape(1, D), beta.reshape(1, D))
    return out


# speedup vs baseline: 22.0348x; 1.0379x over previous
import functools

import jax
import jax.numpy as jnp
from jax import lax
from jax.experimental import pallas as pl
from jax.experimental.pallas import tpu as pltpu
from jax.experimental.pallas import tpu_sc as plsc

N = 10000
E = 320000
D = 128
NE = 8
NGT = 20
H = 2 * D

NP = 10240
NC = 2
NS = 16
NW = NC * NS
EW = E // NW
ROWS = (EW + 127) // 128
HB = NGT * NP
ZB = HB // NS
TRASH = N


def _sc_hist_body(dst_hbm, egt_hbm, zeros_hbm, out_hbm,
                  dst_v, egt_v, idx_v, ones_v, zbuf_v, hist_sh, sem):
    cid = lax.axis_index("c")
    sid = lax.axis_index("s")
    wid = cid * NS + sid

    pltpu.sync_copy(zeros_hbm, zbuf_v)
    pltpu.sync_copy(zbuf_v, hist_sh.at[pl.ds(sid * ZB, ZB)])

    base = wid * EW
    cp_d = pltpu.async_copy(dst_hbm.at[pl.ds(base, EW)], dst_v, sem)
    cp_g = pltpu.async_copy(egt_hbm.at[pl.ds(base, EW)], egt_v, sem)

    for j in range(8):
        ones_v[pl.ds(j * 16, 16)] = jnp.ones((16,), jnp.float32)
    cp_d.wait()
    cp_g.wait()

    def body(r, carry):
        for j in range(8):
            off = r * 128 + j * 16
            d = dst_v[pl.ds(off, 16)]
            g = egt_v[pl.ds(off, 16)]
            idx_v[r, pl.ds(j * 16, 16)] = g * NP + d
        return carry

    lax.fori_loop(0, ROWS - 1, body, 0)
    off = (ROWS - 1) * 128
    idx_v[ROWS - 1, pl.ds(0, 16)] = (egt_v[pl.ds(off, 16)] * NP
                                     + dst_v[pl.ds(off, 16)])
    for j in range(7):
        idx_v[ROWS - 1, pl.ds(16 + j * 16, 16)] = jnp.full(
            (16,), TRASH, jnp.int32)

    plsc.subcore_barrier()
    copies = [pltpu.async_copy(ones_v, hist_sh.at[idx_v.at[r]], sem, add=True)
              for r in range(ROWS)]
    for c in copies:
        c.wait()
    plsc.subcore_barrier()

    off0 = sid * ZB
    pltpu.sync_copy(hist_sh.at[pl.ds(off0, ZB)], zbuf_v)
    pltpu.sync_copy(zbuf_v, out_hbm.at[pl.ds(cid * HB + off0, ZB)])


@functools.cache
def _sc_hist():
    return pl.kernel(
        _sc_hist_body,
        out_type=jax.ShapeDtypeStruct((NC * HB,), jnp.float32),
        mesh=plsc.VectorSubcoreMesh(core_axis_name="c", subcore_axis_name="s",
                                    num_cores=NC, num_subcores=NS),
        scratch_types=[
            pltpu.VMEM((EW,), jnp.int32),
            pltpu.VMEM((EW,), jnp.int32),
            pltpu.VMEM((ROWS, 128), jnp.int32),
            pltpu.VMEM((128,), jnp.float32),
            pltpu.VMEM((ZB,), jnp.float32),
            pltpu.VMEM_SHARED((HB,), jnp.float32),
            pltpu.SemaphoreType.DMA,
        ],
    )


def _tc_body(x_ref, ht_ref, gte_ref, wr_ref, br_ref, w1_ref, b1c_ref,
             w2c_ref, b2_ref, sel_ref, gamma_ref, beta_ref, o_ref):
    x = x_ref[...]
    ct = ht_ref[0:NGT] + ht_ref[NGT:2 * NGT]
    gl_aug = lax.dot_general(
        ct, gte_ref[...], (((0,), (0,)), ((), ())),
        precision=lax.Precision.HIGHEST,
        preferred_element_type=jnp.float32)
    deg = gl_aug[:, NE:NE + 1]
    gate_logits = gl_aug[:, 0:NE] / jnp.maximum(deg, 1.0)
    logits = (jnp.dot(x, wr_ref[...], preferred_element_type=jnp.float32)
              + br_ref[...] + gate_logits)

    iota = lax.broadcasted_iota(jnp.int32, logits.shape, 1)
    m1 = jnp.max(logits, axis=1, keepdims=True)
    idx1 = jnp.min(jnp.where(logits == m1, iota, NE), axis=1, keepdims=True)
    masked = jnp.where(iota == idx1, -jnp.inf, logits)
    m2 = jnp.max(masked, axis=1, keepdims=True)
    idx2 = jnp.min(jnp.where(masked == m2, iota, NE), axis=1, keepdims=True)
    g1 = 1.0 / (1.0 + jnp.exp(m2 - m1))
    w = (jnp.where(iota == idx1, g1, 0.0)
         + jnp.where(iota == idx2, 1.0 - g1, 0.0))

    inv_sqrt2 = 0.7071067811865476
    xb = x.astype(jnp.bfloat16)
    hall = jnp.concatenate(
        [jnp.dot(xb, w1_ref[e], preferred_element_type=jnp.float32)
         for e in range(NE)], axis=1) + b1c_ref[...]
    wrep = jnp.dot(w, sel_ref[...],
                   preferred_element_type=jnp.float32)
    g = (hall * wrep) * (1.0 + lax.erf(hall * inv_sqrt2))
    acc = (jnp.dot(g.astype(jnp.bfloat16), w2c_ref[...],
                   preferred_element_type=jnp.float32)
           + jnp.dot(w, b2_ref[...], preferred_element_type=jnp.float32))

    mu = jnp.mean(acc, axis=1, keepdims=True)
    var = jnp.mean((acc - mu) ** 2, axis=1, keepdims=True)
    o_ref[...] = ((acc - mu) * lax.rsqrt(var + 1e-5) * gamma_ref[...]
                  + beta_ref[...])


def kernel(x, edge_gate_type, edge_index, gate_type_embed, Wr, br,
           W1, b1, W2, b2, gamma, beta):
    dst = edge_index[1].astype(jnp.int32)
    egt = edge_gate_type.astype(jnp.int32)
    zeros = jnp.zeros((ZB,), jnp.float32)

    hist = _sc_hist()(dst, egt, zeros)
    ht = hist.reshape(NC * NGT, NP)

    HH = NE * H
    W1b = W1.astype(jnp.bfloat16)
    b1c = b1.reshape(1, HH)
    W2cb = W2.reshape(HH, D).astype(jnp.bfloat16)
    sel = 0.5 * jnp.repeat(jnp.eye(NE, dtype=jnp.float32), H, axis=1)
    gte_aug = jnp.concatenate(
        [gate_type_embed, jnp.ones((NGT, 1), jnp.float32)], axis=1)

    Bn = 1024
    grid = ((N + Bn - 1) // Bn,)
    out = pl.pallas_call(
        _tc_body,
        grid=grid,
        in_specs=[
            pl.BlockSpec((Bn, D), lambda i: (i, 0)),
            pl.BlockSpec((NC * NGT, Bn), lambda i: (0, i)),
            pl.BlockSpec((NGT, NE + 1), lambda i: (0, 0)),
            pl.BlockSpec((D, NE), lambda i: (0, 0)),
            pl.BlockSpec((1, NE), lambda i: (0, 0)),
            pl.BlockSpec((NE, D, H), lambda i: (0, 0, 0)),
            pl.BlockSpec((1, HH), lambda i: (0, 0)),
            pl.BlockSpec((HH, D), lambda i: (0, 0)),
            pl.BlockSpec((NE, D), lambda i: (0, 0)),
            pl.BlockSpec((NE, HH), lambda i: (0, 0)),
            pl.BlockSpec((1, D), lambda i: (0, 0)),
            pl.BlockSpec((1, D), lambda i: (0, 0)),
        ],
        out_specs=pl.BlockSpec((Bn, D), lambda i: (i, 0)),
        out_shape=jax.ShapeDtypeStruct((N, D), jnp.float32),
        compiler_params=pltpu.CompilerParams(
            dimension_semantics=("arbitrary",)),
    )(x, ht, gte_aug, Wr, br.reshape(1, NE), W1b, b1c,
      W2cb, b2, sel, gamma.reshape(1, D), beta.reshape(1, D))
    return out


# speedup vs baseline: 22.2649x; 1.0104x over previous
import functools

import jax
import jax.numpy as jnp
from jax import lax
from jax.experimental import pallas as pl
from jax.experimental.pallas import tpu as pltpu
from jax.experimental.pallas import tpu_sc as plsc

N = 10000
E = 320000
D = 128
NE = 8
NGT = 20
H = 2 * D

NP = 10240
NC = 2
NS = 16
NW = NC * NS
EW = E // NW
ROWS = (EW + 127) // 128
HB = NGT * NP
ZB = HB // NS
TRASH = N


def _sc_hist_body(dst_hbm, egt_hbm, zeros_hbm, out_hbm,
                  dst_v, egt_v, idx_v, ones_v, zbuf_v, hist_sh, sem):
    cid = lax.axis_index("c")
    sid = lax.axis_index("s")
    wid = cid * NS + sid

    pltpu.sync_copy(zeros_hbm, zbuf_v)
    pltpu.sync_copy(zbuf_v, hist_sh.at[pl.ds(sid * ZB, ZB)])

    base = wid * EW
    cp_d = pltpu.async_copy(dst_hbm.at[pl.ds(base, EW)], dst_v, sem)
    cp_g = pltpu.async_copy(egt_hbm.at[pl.ds(base, EW)], egt_v, sem)

    for j in range(8):
        ones_v[pl.ds(j * 16, 16)] = jnp.ones((16,), jnp.float32)
    cp_d.wait()
    cp_g.wait()

    def body(r, carry):
        for j in range(8):
            off = r * 128 + j * 16
            d = dst_v[pl.ds(off, 16)]
            g = egt_v[pl.ds(off, 16)]
            idx_v[r, pl.ds(j * 16, 16)] = g * NP + d
        return carry

    lax.fori_loop(0, ROWS - 1, body, 0)
    off = (ROWS - 1) * 128
    idx_v[ROWS - 1, pl.ds(0, 16)] = (egt_v[pl.ds(off, 16)] * NP
                                     + dst_v[pl.ds(off, 16)])
    for j in range(7):
        idx_v[ROWS - 1, pl.ds(16 + j * 16, 16)] = jnp.full(
            (16,), TRASH, jnp.int32)

    plsc.subcore_barrier()
    copies = [pltpu.async_copy(ones_v, hist_sh.at[idx_v.at[r]], sem, add=True)
              for r in range(ROWS)]
    for c in copies:
        c.wait()
    plsc.subcore_barrier()

    off0 = sid * ZB
    pltpu.sync_copy(hist_sh.at[pl.ds(off0, ZB)], zbuf_v)
    pltpu.sync_copy(zbuf_v, out_hbm.at[pl.ds(cid * HB + off0, ZB)])


@functools.cache
def _sc_hist():
    return pl.kernel(
        _sc_hist_body,
        out_type=jax.ShapeDtypeStruct((NC * HB,), jnp.float32),
        mesh=plsc.VectorSubcoreMesh(core_axis_name="c", subcore_axis_name="s",
                                    num_cores=NC, num_subcores=NS),
        scratch_types=[
            pltpu.VMEM((EW,), jnp.int32),
            pltpu.VMEM((EW,), jnp.int32),
            pltpu.VMEM((ROWS, 128), jnp.int32),
            pltpu.VMEM((128,), jnp.float32),
            pltpu.VMEM((ZB,), jnp.float32),
            pltpu.VMEM_SHARED((HB,), jnp.float32),
            pltpu.SemaphoreType.DMA,
        ],
    )


def _tc_body(x_ref, ht_ref, gte_ref, wr_ref, br_ref, w1_ref, b1c_ref,
             w2c_ref, b2_ref, sel_ref, gamma_ref, beta_ref, o_ref):
    x = x_ref[...]
    ct = ht_ref[0:NGT] + ht_ref[NGT:2 * NGT]
    gl_aug = lax.dot_general(
        ct, gte_ref[...], (((0,), (0,)), ((), ())),
        precision=lax.Precision.HIGHEST,
        preferred_element_type=jnp.float32)
    deg = gl_aug[:, NE:NE + 1]
    gate_logits = gl_aug[:, 0:NE] / jnp.maximum(deg, 1.0)
    logits = (jnp.dot(x, wr_ref[...], preferred_element_type=jnp.float32)
              + br_ref[...] + gate_logits)

    iota = lax.broadcasted_iota(jnp.int32, logits.shape, 1)
    m1 = jnp.max(logits, axis=1, keepdims=True)
    idx1 = jnp.min(jnp.where(logits == m1, iota, NE), axis=1, keepdims=True)
    masked = jnp.where(iota == idx1, -jnp.inf, logits)
    m2 = jnp.max(masked, axis=1, keepdims=True)
    idx2 = jnp.min(jnp.where(masked == m2, iota, NE), axis=1, keepdims=True)
    g1 = 1.0 / (1.0 + jnp.exp(m2 - m1))
    w = (jnp.where(iota == idx1, g1, 0.0)
         + jnp.where(iota == idx2, 1.0 - g1, 0.0))

    xb = x.astype(jnp.bfloat16)
    hc = jnp.concatenate(
        [jnp.dot(xb, w1_ref[e], preferred_element_type=jnp.float32)
         for e in range(NE)], axis=1) + b1c_ref[...]
    wrep = jnp.dot(w, sel_ref[...],
                   preferred_element_type=jnp.float32)
    g = (hc * wrep) * (1.0 + lax.erf(hc))
    acc = (jnp.dot(g.astype(jnp.bfloat16), w2c_ref[...],
                   preferred_element_type=jnp.float32)
           + jnp.dot(w, b2_ref[...], preferred_element_type=jnp.float32))

    mu = jnp.mean(acc, axis=1, keepdims=True)
    var = jnp.mean((acc - mu) ** 2, axis=1, keepdims=True)
    o_ref[...] = ((acc - mu) * lax.rsqrt(var + 1e-5) * gamma_ref[...]
                  + beta_ref[...])


def kernel(x, edge_gate_type, edge_index, gate_type_embed, Wr, br,
           W1, b1, W2, b2, gamma, beta):
    dst = edge_index[1].astype(jnp.int32)
    egt = edge_gate_type.astype(jnp.int32)
    zeros = jnp.zeros((ZB,), jnp.float32)

    hist = _sc_hist()(dst, egt, zeros)
    ht = hist.reshape(NC * NGT, NP)

    HH = NE * H
    inv_sqrt2 = 0.7071067811865476
    W1b = (W1 * inv_sqrt2).astype(jnp.bfloat16)
    b1c = (b1 * inv_sqrt2).reshape(1, HH)
    W2cb = W2.reshape(HH, D).astype(jnp.bfloat16)
    sel = (0.5 / inv_sqrt2) * jnp.repeat(jnp.eye(NE, dtype=jnp.float32),
                                         H, axis=1)
    gte_aug = jnp.concatenate(
        [gate_type_embed, jnp.ones((NGT, 1), jnp.float32)], axis=1)

    Bn = 1024
    grid = ((N + Bn - 1) // Bn,)
    out = pl.pallas_call(
        _tc_body,
        grid=grid,
        in_specs=[
            pl.BlockSpec((Bn, D), lambda i: (i, 0)),
            pl.BlockSpec((NC * NGT, Bn), lambda i: (0, i)),
            pl.BlockSpec((NGT, NE + 1), lambda i: (0, 0)),
            pl.BlockSpec((D, NE), lambda i: (0, 0)),
            pl.BlockSpec((1, NE), lambda i: (0, 0)),
            pl.BlockSpec((NE, D, H), lambda i: (0, 0, 0)),
            pl.BlockSpec((1, HH), lambda i: (0, 0)),
            pl.BlockSpec((HH, D), lambda i: (0, 0)),
            pl.BlockSpec((NE, D), lambda i: (0, 0)),
            pl.BlockSpec((NE, HH), lambda i: (0, 0)),
            pl.BlockSpec((1, D), lambda i: (0, 0)),
            pl.BlockSpec((1, D), lambda i: (0, 0)),
        ],
        out_specs=pl.BlockSpec((Bn, D), lambda i: (i, 0)),
        out_shape=jax.ShapeDtypeStruct((N, D), jnp.float32),
        compiler_params=pltpu.CompilerParams(
            dimension_semantics=("arbitrary",)),
    )(x, ht, gte_aug, Wr, br.reshape(1, NE), W1b, b1c,
      W2cb, b2, sel, gamma.reshape(1, D), beta.reshape(1, D))
    return out
